# baseline (speedup 1.0000x reference)
import jax
import jax.numpy as jnp
from jax import lax
from jax.experimental import pallas as pl
from jax.experimental.pallas import tpu as pltpu
from jax.experimental.pallas import tpu_sc as plsc

N = 100000
D = 128
P = 50000
L = 16
NC, NS = 2, 16
NW = NC * NS
CHUNK = 128
BW = 1664
NCHUNK = BW // CHUNK
PAD = NW * BW


def _body(idx_a, idx_b, t_aig, t_mig, t_xmg, t_xag, out,
          idxa_v, idxb_v, rows1, rows2, acc_v, sem1, sem2):
    wid = lax.axis_index("s") * NC + lax.axis_index("c")
    base = wid * BW

    acc_v[...] = jnp.zeros((L,), jnp.float32)

    for p, tab2 in enumerate((t_mig, t_xmg, t_xag)):
        pltpu.sync_copy(idx_a.at[p, wid], idxa_v)
        pltpu.sync_copy(idx_b.at[p, wid], idxb_v)

        @pl.loop(0, NCHUNK)
        def _chunk(cc):
            cp1 = pltpu.async_copy(t_aig.at[idxa_v.at[cc]], rows1, sem1)
            cp2 = pltpu.async_copy(tab2.at[idxb_v.at[cc]], rows2, sem2)
            cp1.wait()
            cp2.wait()

            row0 = base + cc * CHUNK

            @pl.loop(0, CHUNK)
            def _row(r):
                c = jnp.zeros((L,), jnp.float32)
                for k in range(D // L):
                    a = rows1[r, pl.ds(k * L, L)]
                    b = rows2[r, pl.ds(k * L, L)]
                    c = c + jnp.abs(a - b)
                valid = (row0 + r) < P
                acc_v[...] = acc_v[...] + jnp.where(valid, c, 0.0)

    pltpu.sync_copy(acc_v, out.at[wid])


@jax.jit
def _run(idx_a, idx_b, t_aig, t_mig, t_xmg, t_xag):
    mesh = plsc.VectorSubcoreMesh(core_axis_name="c", subcore_axis_name="s")
    f = pl.kernel(
        _body,
        out_type=jax.ShapeDtypeStruct((NW, L), jnp.float32),
        mesh=mesh,
        scratch_types=[
            pltpu.VMEM((NCHUNK, CHUNK), jnp.int32),
            pltpu.VMEM((NCHUNK, CHUNK), jnp.int32),
            pltpu.VMEM((CHUNK, D), jnp.float32),
            pltpu.VMEM((CHUNK, D), jnp.float32),
            pltpu.VMEM((L,), jnp.float32),
            pltpu.SemaphoreType.DMA,
            pltpu.SemaphoreType.DMA,
        ],
    )
    partials = f(idx_a, idx_b, t_aig, t_mig, t_xmg, t_xag)
    return jnp.sum(partials) / jnp.float32(P * D)


def kernel(aig_hf, mig_hf, xmg_hf, xag_hf,
           aig_mig_equ, mig_aig_equ,
           aig_xmg_equ, xmg_aig_equ,
           aig_xag_equ, xag_aig_equ):
    idx_a = jnp.stack([aig_mig_equ, aig_xmg_equ, aig_xag_equ]).astype(jnp.int32)
    idx_b = jnp.stack([mig_aig_equ, xmg_aig_equ, xag_aig_equ]).astype(jnp.int32)
    idx_a = jnp.pad(idx_a, ((0, 0), (0, PAD - P))).reshape(3, NW, NCHUNK, CHUNK)
    idx_b = jnp.pad(idx_b, ((0, 0), (0, PAD - P))).reshape(3, NW, NCHUNK, CHUNK)
    return _run(idx_a, idx_b, aig_hf, mig_hf, xmg_hf, xag_hf)


# speedup vs baseline: 3.2964x; 3.2964x over previous
import jax
import jax.numpy as jnp
from jax import lax
from jax.experimental import pallas as pl
from jax.experimental.pallas import tpu as pltpu
from jax.experimental.pallas import tpu_sc as plsc

N = 100000
D = 128
P = 50000
L = 16
NC, NS = 2, 16
NW = NC * NS
CHUNK = 112
NCHUNK = 14
BW = CHUNK * NCHUNK
PAD = NW * BW


def _body(idx_a, idx_b, t_aig, t_mig, t_xmg, t_xag, out,
          idxa_v, idxb_v, rA1, rA2, rB1, rB2, acc_v, semA, semB):
    wid = lax.axis_index("s") * NC + lax.axis_index("c")
    base = wid * BW

    pltpu.sync_copy(idx_a.at[wid], idxa_v)
    pltpu.sync_copy(idx_b.at[wid], idxb_v)

    def start(p, cc, r1, r2, sem, tab2):
        pltpu.async_copy(t_aig.at[idxa_v.at[p, cc]], r1, sem)
        pltpu.async_copy(tab2.at[idxb_v.at[p, cc]], r2, sem)

    def drain(r1, r2, sem):
        pltpu.make_async_copy(t_aig.at[idxa_v.at[0, 0]], r1, sem).wait()
        pltpu.make_async_copy(t_aig.at[idxb_v.at[0, 0]], r2, sem).wait()

    def compute(r1, r2, acc):
        @pl.loop(0, CHUNK, init_carry=acc, unroll=4)
        def _rows(r, acc):
            a0, a1 = acc
            t = [jnp.abs(r1[r, pl.ds(k * L, L)] - r2[r, pl.ds(k * L, L)])
                 for k in range(D // L)]
            c = ((t[0] + t[1]) + (t[2] + t[3])) + ((t[4] + t[5]) + (t[6] + t[7]))
            return (a1, a0 + c)
        return _rows

    acc = (jnp.zeros((L,), jnp.float32), jnp.zeros((L,), jnp.float32))

    for p, tab2 in enumerate((t_mig, t_xmg, t_xag)):
        start(p, 0, rA1, rA2, semA, tab2)

        @pl.loop(0, (NCHUNK - 2) // 2, init_carry=acc)
        def _chunks(t, acc, p=p, tab2=tab2):
            cc = 2 * t
            start(p, cc + 1, rB1, rB2, semB, tab2)
            drain(rA1, rA2, semA)
            acc = compute(rA1, rA2, acc)
            start(p, cc + 2, rA1, rA2, semA, tab2)
            drain(rB1, rB2, semB)
            acc = compute(rB1, rB2, acc)
            return acc

        acc = _chunks
        start(p, NCHUNK - 1, rB1, rB2, semB, tab2)
        drain(rA1, rA2, semA)
        acc = compute(rA1, rA2, acc)
        drain(rB1, rB2, semB)
        acc = compute(rB1, rB2, acc)

        f_pad = jnp.clip(base + BW - P, 0, BW).astype(jnp.float32)
        t = [jnp.abs(rB1[CHUNK - 1, pl.ds(k * L, L)]
                     - rB2[CHUNK - 1, pl.ds(k * L, L)])
             for k in range(D // L)]
        v0 = ((t[0] + t[1]) + (t[2] + t[3])) + ((t[4] + t[5]) + (t[6] + t[7]))
        acc = (acc[0] - f_pad * v0, acc[1])

    acc_v[...] = acc[0] + acc[1]
    pltpu.sync_copy(acc_v, out.at[wid])


@jax.jit
def _run(idx_a, idx_b, t_aig, t_mig, t_xmg, t_xag):
    mesh = plsc.VectorSubcoreMesh(core_axis_name="c", subcore_axis_name="s")
    f = pl.kernel(
        _body,
        out_type=jax.ShapeDtypeStruct((NW, L), jnp.float32),
        mesh=mesh,
        scratch_types=[
            pltpu.VMEM((3, NCHUNK, CHUNK), jnp.int32),
            pltpu.VMEM((3, NCHUNK, CHUNK), jnp.int32),
            pltpu.VMEM((CHUNK, D), jnp.float32),
            pltpu.VMEM((CHUNK, D), jnp.float32),
            pltpu.VMEM((CHUNK, D), jnp.float32),
            pltpu.VMEM((CHUNK, D), jnp.float32),
            pltpu.VMEM((L,), jnp.float32),
            pltpu.SemaphoreType.DMA,
            pltpu.SemaphoreType.DMA,
        ],
    )
    partials = f(idx_a, idx_b, t_aig, t_mig, t_xmg, t_xag)
    return jnp.sum(partials) / jnp.float32(P * D)


def kernel(aig_hf, mig_hf, xmg_hf, xag_hf,
           aig_mig_equ, mig_aig_equ,
           aig_xmg_equ, xmg_aig_equ,
           aig_xag_equ, xag_aig_equ):
    idx_a = jnp.stack([aig_mig_equ, aig_xmg_equ, aig_xag_equ]).astype(jnp.int32)
    idx_b = jnp.stack([mig_aig_equ, xmg_aig_equ, xag_aig_equ]).astype(jnp.int32)
    idx_a = (jnp.pad(idx_a, ((0, 0), (0, PAD - P)))
             .reshape(3, NW, NCHUNK, CHUNK).transpose(1, 0, 2, 3))
    idx_b = (jnp.pad(idx_b, ((0, 0), (0, PAD - P)))
             .reshape(3, NW, NCHUNK, CHUNK).transpose(1, 0, 2, 3))
    return _run(idx_a, idx_b, aig_hf, mig_hf, xmg_hf, xag_hf)
